# baseline (speedup 1.0000x reference)
import functools

import jax
import jax.numpy as jnp
from jax import lax
from jax.experimental import pallas as pl
from jax.experimental.pallas import tpu as pltpu
from jax.experimental.pallas import tpu_sc as plsc

M = 8
R = 8
V = 100000
D = 64
B = 1024
T = 50
LORA_ALPHA = 16.0
SCALING = LORA_ALPHA / R

N = B * T
NC, NS = 2, 16
NW = NC * NS
B_PER_W = N // NW
HALF = B_PER_W // 2
CH = 128


def _tc_prep(weight, lora_a_flat):
    vt = 1024
    grid = (pl.cdiv(V, vt),)

    def body(w_ref, a_ref, out_ref):
        out_ref[:, :D] = w_ref[...]
        out_ref[:, D:] = jnp.transpose(a_ref[...], (1, 0))

    return pl.pallas_call(
        body,
        grid=grid,
        in_specs=[
            pl.BlockSpec((vt, D), lambda i: (i, 0)),
            pl.BlockSpec((M * R, vt), lambda i: (0, i)),
        ],
        out_specs=pl.BlockSpec((vt, 2 * D), lambda i: (i, 0)),
        out_shape=jax.ShapeDtypeStruct((V, 2 * D), jnp.float32),
    )(weight, lora_a_flat)


def _sc_gather(idx_flat, wc):
    mesh = plsc.VectorSubcoreMesh(core_axis_name="c", subcore_axis_name="s")

    @functools.partial(
        pl.kernel,
        mesh=mesh,
        compiler_params=pltpu.CompilerParams(use_tc_tiling_on_sc=True),
        out_type=jax.ShapeDtypeStruct((N, 2 * D), jnp.float32),
        scratch_types=[
            pltpu.VMEM((B_PER_W,), jnp.int32),
            pltpu.VMEM((HALF, 2 * D), jnp.float32),
            pltpu.SemaphoreType.DMA,
        ],
    )
    def gather_kernel(idx_hbm, wc_hbm, g_hbm, idx_v, rows_v, sem):
        wid = lax.axis_index("s") * NC + lax.axis_index("c")
        base = wid * B_PER_W
        pltpu.sync_copy(idx_hbm.at[pl.ds(base, B_PER_W)], idx_v)
        for h in range(2):
            copies = []
            for lo in range(0, HALF, CH):
                sz = min(CH, HALF - lo)
                copies.append(pltpu.async_copy(
                    wc_hbm.at[idx_v.at[pl.ds(h * HALF + lo, sz)]],
                    rows_v.at[pl.ds(lo, sz)],
                    sem,
                ))
            for cp in copies:
                cp.wait()
            pltpu.sync_copy(rows_v, g_hbm.at[pl.ds(base + h * HALF, HALF)])

    return gather_kernel(idx_flat, wc)


def _tc_combine(g, qf):
    n2 = N // 2
    tn = 2048
    grid = (N // tn,)

    def body(g_ref, qf_ref, out_ref):
        g2 = g_ref[...].reshape(tn // 2, 4 * D)
        for m in range(M):
            out_ref[m] = jnp.dot(g2, qf_ref[m],
                                 preferred_element_type=jnp.float32)

    return pl.pallas_call(
        body,
        grid=grid,
        in_specs=[
            pl.BlockSpec((tn, 2 * D), lambda i: (i, 0)),
            pl.BlockSpec((M, 4 * D, 2 * D), lambda i: (0, 0, 0)),
        ],
        out_specs=pl.BlockSpec((M, tn // 2, 2 * D), lambda i: (0, i, 0)),
        out_shape=jax.ShapeDtypeStruct((M, n2, 2 * D), jnp.float32),
    )(g, qf)


def kernel(x, weight, lora_A, lora_B):
    idx_flat = x.reshape(N)
    wc = _tc_prep(weight, lora_A.reshape(M * R, V))
    g = _sc_gather(idx_flat, wc)

    p = SCALING * jnp.transpose(lora_B, (0, 2, 1))
    p_tiled = jnp.tile(p, (1, M, 1))
    sel = (jnp.arange(M * R)[None, :, None] // R
           == jnp.arange(M)[:, None, None])
    p_big = jnp.where(sel, p_tiled, 0.0)
    eye = jnp.broadcast_to(jnp.eye(D, dtype=jnp.float32), (M, D, D))
    u = jnp.concatenate([eye, p_big], axis=1)
    qf = jax.vmap(lambda um: jnp.kron(jnp.eye(2, dtype=um.dtype), um))(u)

    out2 = _tc_combine(g, qf)
    return out2.reshape(M, B, T, D)


# speedup vs baseline: 2.4044x; 2.4044x over previous
import functools

import jax
import jax.numpy as jnp
from jax import lax
from jax.experimental import pallas as pl
from jax.experimental.pallas import tpu as pltpu
from jax.experimental.pallas import tpu_sc as plsc

M = 8
R = 8
V = 100000
D = 64
B = 1024
T = 50
LORA_ALPHA = 16.0
SCALING = LORA_ALPHA / R

N = B * T
NC, NS = 2, 16
NW = NC * NS
B_PER_W = N // NW
HALF = B_PER_W // 2
CH = 128


def _tc_prep(weight, lora_a_flat):
    vt = 1024
    grid = (pl.cdiv(V, vt),)

    def body(w_ref, a_ref, out_ref):
        out_ref[:, :D] = jnp.transpose(w_ref[...], (1, 0))
        out_ref[:, D:] = jnp.transpose(a_ref[...], (1, 0))

    return pl.pallas_call(
        body,
        grid=grid,
        in_specs=[
            pl.BlockSpec((D, vt), lambda i: (0, i)),
            pl.BlockSpec((M * R, vt), lambda i: (0, i)),
        ],
        out_specs=pl.BlockSpec((vt, 2 * D), lambda i: (i, 0)),
        out_shape=jax.ShapeDtypeStruct((V, 2 * D), jnp.float32),
    )(weight, lora_a_flat)


def _sc_gather(idx_flat, wc):
    mesh = plsc.VectorSubcoreMesh(core_axis_name="c", subcore_axis_name="s")

    @functools.partial(
        pl.kernel,
        mesh=mesh,
        compiler_params=pltpu.CompilerParams(use_tc_tiling_on_sc=True),
        out_type=jax.ShapeDtypeStruct((N, 2 * D), jnp.float32),
        scratch_types=[
            pltpu.VMEM((B_PER_W,), jnp.int32),
            pltpu.VMEM((HALF, 2 * D), jnp.float32),
            pltpu.SemaphoreType.DMA,
        ],
    )
    def gather_kernel(idx_hbm, wc_hbm, g_hbm, idx_v, rows_v, sem):
        wid = lax.axis_index("s") * NC + lax.axis_index("c")
        base = wid * B_PER_W
        pltpu.sync_copy(idx_hbm.at[pl.ds(base, B_PER_W)], idx_v)
        for h in range(2):
            copies = []
            for lo in range(0, HALF, CH):
                sz = min(CH, HALF - lo)
                copies.append(pltpu.async_copy(
                    wc_hbm.at[idx_v.at[pl.ds(h * HALF + lo, sz)]],
                    rows_v.at[pl.ds(lo, sz)],
                    sem,
                ))
            for cp in copies:
                cp.wait()
            pltpu.sync_copy(rows_v, g_hbm.at[pl.ds(base + h * HALF, HALF)])

    return gather_kernel(idx_flat, wc)


def _tc_combine(g3, uft):
    def body(g_ref, u_ref, out_ref):
        g = g_ref[0]
        for m in range(M):
            out_ref[m, 0] = lax.dot_general(
                u_ref[m], g, (((1,), (1,)), ((), ())),
                preferred_element_type=jnp.float32)

    return pl.pallas_call(
        body,
        grid=(T,),
        in_specs=[
            pl.BlockSpec((1, B, 2 * D), lambda i: (i, 0, 0)),
            pl.BlockSpec((M, D, 2 * D), lambda i: (0, 0, 0)),
        ],
        out_specs=pl.BlockSpec((M, 1, D, B), lambda i: (0, i, 0, 0)),
        out_shape=jax.ShapeDtypeStruct((M, T, D, B), jnp.float32),
    )(g3, uft)


def kernel(x, weight, lora_A, lora_B):
    idx_flat = jnp.swapaxes(x, 0, 1).reshape(N)
    wc = _tc_prep(jnp.transpose(weight), lora_A.reshape(M * R, V))
    g = _sc_gather(idx_flat, wc)

    p = SCALING * jnp.transpose(lora_B, (0, 2, 1))
    p_tiled = jnp.tile(p, (1, M, 1))
    sel = (jnp.arange(M * R)[None, :, None] // R
           == jnp.arange(M)[:, None, None])
    p_big = jnp.where(sel, p_tiled, 0.0)
    eye = jnp.broadcast_to(jnp.eye(D, dtype=jnp.float32), (M, D, D))
    u = jnp.concatenate([eye, p_big], axis=1)
    uft = jnp.transpose(u, (0, 2, 1))

    out_phys = _tc_combine(g.reshape(T, B, 2 * D), uft)
    return jnp.transpose(out_phys, (0, 3, 1, 2))


# speedup vs baseline: 2.7125x; 1.1282x over previous
import functools

import jax
import jax.numpy as jnp
from jax import lax
from jax.experimental import pallas as pl
from jax.experimental.pallas import tpu as pltpu
from jax.experimental.pallas import tpu_sc as plsc

M = 8
R = 8
V = 100000
D = 64
B = 1024
T = 50
LORA_ALPHA = 16.0
SCALING = LORA_ALPHA / R

N = B * T
NC, NS = 2, 16
NW = NC * NS
B_PER_W = N // NW
HALF = B_PER_W // 2
CH = 128


def _tc_prep(weight, lora_a_flat):
    vt = 1024
    grid = (pl.cdiv(V, vt),)

    def body(w_ref, a_ref, eye_ref, out_ref):
        stacked = jnp.concatenate([w_ref[...], a_ref[...]], axis=0)
        out_ref[...] = lax.dot_general(
            stacked, eye_ref[...], (((0,), (0,)), ((), ())),
            preferred_element_type=jnp.float32)

    return pl.pallas_call(
        body,
        grid=grid,
        in_specs=[
            pl.BlockSpec((D, vt), lambda i: (0, i)),
            pl.BlockSpec((M * R, vt), lambda i: (0, i)),
            pl.BlockSpec((2 * D, 2 * D), lambda i: (0, 0)),
        ],
        out_specs=pl.BlockSpec((vt, 2 * D), lambda i: (i, 0)),
        out_shape=jax.ShapeDtypeStruct((V, 2 * D), jnp.float32),
    )(weight, lora_a_flat, jnp.eye(2 * D, dtype=jnp.float32))


def _sc_gather(idx_flat, wc):
    mesh = plsc.VectorSubcoreMesh(core_axis_name="c", subcore_axis_name="s")

    @functools.partial(
        pl.kernel,
        mesh=mesh,
        compiler_params=pltpu.CompilerParams(use_tc_tiling_on_sc=True),
        out_type=jax.ShapeDtypeStruct((N, 2 * D), jnp.float32),
        scratch_types=[
            pltpu.VMEM((B_PER_W,), jnp.int32),
            pltpu.VMEM((HALF, 2 * D), jnp.float32),
            pltpu.SemaphoreType.DMA,
        ],
    )
    def gather_kernel(idx_hbm, wc_hbm, g_hbm, idx_v, rows_v, sem):
        wid = lax.axis_index("s") * NC + lax.axis_index("c")
        base = wid * B_PER_W
        pltpu.sync_copy(idx_hbm.at[pl.ds(base, B_PER_W)], idx_v)
        for h in range(2):
            copies = []
            for lo in range(0, HALF, CH):
                sz = min(CH, HALF - lo)
                copies.append(pltpu.async_copy(
                    wc_hbm.at[idx_v.at[pl.ds(h * HALF + lo, sz)]],
                    rows_v.at[pl.ds(lo, sz)],
                    sem,
                ))
            for cp in copies:
                cp.wait()
            pltpu.sync_copy(rows_v, g_hbm.at[pl.ds(base + h * HALF, HALF)])

    return gather_kernel(idx_flat, wc)


def _tc_combine(g3, uft):
    tt = 2

    def body(g_ref, u_ref, out_ref):
        for t in range(tt):
            g = g_ref[t]
            for m in range(M):
                out_ref[m, t] = lax.dot_general(
                    u_ref[m], g, (((1,), (1,)), ((), ())),
                    preferred_element_type=jnp.float32)

    return pl.pallas_call(
        body,
        grid=(T // tt,),
        in_specs=[
            pl.BlockSpec((tt, B, 2 * D), lambda i: (i, 0, 0)),
            pl.BlockSpec((M, D, 2 * D), lambda i: (0, 0, 0)),
        ],
        out_specs=pl.BlockSpec((M, tt, D, B), lambda i: (0, i, 0, 0)),
        out_shape=jax.ShapeDtypeStruct((M, T, D, B), jnp.float32),
    )(g3, uft)


def kernel(x, weight, lora_A, lora_B):
    idx_flat = jnp.swapaxes(x, 0, 1).reshape(N)
    wc = _tc_prep(jnp.transpose(weight), lora_A.reshape(M * R, V))
    g = _sc_gather(idx_flat, wc)

    p = SCALING * jnp.transpose(lora_B, (0, 2, 1))
    p_tiled = jnp.tile(p, (1, M, 1))
    sel = (jnp.arange(M * R)[None, :, None] // R
           == jnp.arange(M)[:, None, None])
    p_big = jnp.where(sel, p_tiled, 0.0)
    eye = jnp.broadcast_to(jnp.eye(D, dtype=jnp.float32), (M, D, D))
    u = jnp.concatenate([eye, p_big], axis=1)
    uft = jnp.transpose(u, (0, 2, 1))

    out_phys = _tc_combine(g.reshape(T, B, 2 * D), uft)
    return jnp.transpose(out_phys, (0, 3, 1, 2))


# speedup vs baseline: 3.3713x; 1.2429x over previous
import functools

import jax
import jax.numpy as jnp
from jax import lax
from jax.experimental import pallas as pl
from jax.experimental.pallas import tpu as pltpu
from jax.experimental.pallas import tpu_sc as plsc

M = 8
R = 8
V = 100000
D = 64
B = 1024
T = 50
LORA_ALPHA = 16.0
SCALING = LORA_ALPHA / R

N = B * T
NC, NS = 2, 16
NW = NC * NS
B_PER_W = N // NW
HALF = B_PER_W // 2
CH = 128


def _tc_prep(weight, lora_a_flat):
    vt = 2048
    grid = (pl.cdiv(V, vt),)

    def body(w_ref, a_ref, eye_ref, out_ref):
        stacked = jnp.concatenate([w_ref[...], a_ref[...]], axis=0)
        out_ref[...] = lax.dot_general(
            stacked, eye_ref[...], (((0,), (0,)), ((), ())),
            preferred_element_type=jnp.float32)

    return pl.pallas_call(
        body,
        grid=grid,
        in_specs=[
            pl.BlockSpec((D, vt), lambda i: (0, i)),
            pl.BlockSpec((M * R, vt), lambda i: (0, i)),
            pl.BlockSpec((2 * D, 2 * D), lambda i: (0, 0)),
        ],
        out_specs=pl.BlockSpec((vt, 2 * D), lambda i: (i, 0)),
        out_shape=jax.ShapeDtypeStruct((V, 2 * D), jnp.float32),
    )(weight, lora_a_flat, jnp.eye(2 * D, dtype=jnp.float32))


def _sc_gather(idx_flat, wc):
    mesh = plsc.VectorSubcoreMesh(core_axis_name="c", subcore_axis_name="s")

    @functools.partial(
        pl.kernel,
        mesh=mesh,
        compiler_params=pltpu.CompilerParams(use_tc_tiling_on_sc=True),
        out_type=jax.ShapeDtypeStruct((N, 2 * D), jnp.float32),
        scratch_types=[
            pltpu.VMEM((B_PER_W,), jnp.int32),
            pltpu.VMEM((HALF, 2 * D), jnp.float32),
            pltpu.SemaphoreType.DMA,
        ],
    )
    def gather_kernel(idx_hbm, wc_hbm, g_hbm, idx_v, rows_v, sem):
        wid = lax.axis_index("s") * NC + lax.axis_index("c")
        base = wid * B_PER_W
        pltpu.sync_copy(idx_hbm.at[pl.ds(base, B_PER_W)], idx_v)
        for h in range(2):
            copies = []
            for lo in range(0, HALF, CH):
                sz = min(CH, HALF - lo)
                copies.append(pltpu.async_copy(
                    wc_hbm.at[idx_v.at[pl.ds(h * HALF + lo, sz)]],
                    rows_v.at[pl.ds(lo, sz)],
                    sem,
                ))
            for cp in copies:
                cp.wait()
            pltpu.sync_copy(rows_v, g_hbm.at[pl.ds(base + h * HALF, HALF)])

    return gather_kernel(idx_flat, wc)


def _tc_combine(g3, uft):
    tt = 5

    def body(g_ref, u_ref, out_ref):
        for t in range(tt):
            g = g_ref[t]
            for m in range(M):
                out_ref[m, t] = lax.dot_general(
                    u_ref[m], g, (((1,), (1,)), ((), ())),
                    preferred_element_type=jnp.float32)

    return pl.pallas_call(
        body,
        grid=(T // tt,),
        in_specs=[
            pl.BlockSpec((tt, B, 2 * D), lambda i: (i, 0, 0)),
            pl.BlockSpec((M, D, 2 * D), lambda i: (0, 0, 0)),
        ],
        out_specs=pl.BlockSpec((M, tt, D, B), lambda i: (0, i, 0, 0)),
        out_shape=jax.ShapeDtypeStruct((M, T, D, B), jnp.float32),
    )(g3, uft)


def kernel(x, weight, lora_A, lora_B):
    idx_flat = jnp.swapaxes(x, 0, 1).reshape(N)
    wc = _tc_prep(jnp.transpose(weight), lora_A.reshape(M * R, V))
    g = _sc_gather(idx_flat, wc)

    p = SCALING * jnp.transpose(lora_B, (0, 2, 1))
    p_tiled = jnp.tile(p, (1, M, 1))
    sel = (jnp.arange(M * R)[None, :, None] // R
           == jnp.arange(M)[:, None, None])
    p_big = jnp.where(sel, p_tiled, 0.0)
    eye = jnp.broadcast_to(jnp.eye(D, dtype=jnp.float32), (M, D, D))
    u = jnp.concatenate([eye, p_big], axis=1)
    uft = jnp.transpose(u, (0, 2, 1))

    out_phys = _tc_combine(g.reshape(T, B, 2 * D), uft)
    return jnp.transpose(out_phys, (0, 3, 1, 2))


# speedup vs baseline: 3.7865x; 1.1232x over previous
import functools

import jax
import jax.numpy as jnp
from jax import lax
from jax.experimental import pallas as pl
from jax.experimental.pallas import tpu as pltpu
from jax.experimental.pallas import tpu_sc as plsc

M = 8
R = 8
V = 100000
D = 64
B = 1024
T = 50
LORA_ALPHA = 16.0
SCALING = LORA_ALPHA / R

N = B * T
NC, NS = 2, 16
NW = NC * NS
B_PER_W = N // NW
HALF = B_PER_W // 2
CH = 128


def _tc_prep(weight, lora_a_flat):
    vt = 4096
    grid = (pl.cdiv(V, vt),)

    def body(w_ref, a_ref, eye_ref, out_ref):
        stacked = jnp.concatenate([w_ref[...], a_ref[...]], axis=0)
        out_ref[...] = lax.dot_general(
            stacked, eye_ref[...], (((0,), (0,)), ((), ())),
            preferred_element_type=jnp.float32)

    return pl.pallas_call(
        body,
        grid=grid,
        in_specs=[
            pl.BlockSpec((D, vt), lambda i: (0, i)),
            pl.BlockSpec((M * R, vt), lambda i: (0, i)),
            pl.BlockSpec((2 * D, 2 * D), lambda i: (0, 0)),
        ],
        out_specs=pl.BlockSpec((vt, 2 * D), lambda i: (i, 0)),
        out_shape=jax.ShapeDtypeStruct((V, 2 * D), jnp.float32),
    )(weight, lora_a_flat, jnp.eye(2 * D, dtype=jnp.float32))


def _sc_gather(idx_flat, wc):
    mesh = plsc.VectorSubcoreMesh(core_axis_name="c", subcore_axis_name="s")

    @functools.partial(
        pl.kernel,
        mesh=mesh,
        compiler_params=pltpu.CompilerParams(use_tc_tiling_on_sc=True),
        out_type=jax.ShapeDtypeStruct((N, 2 * D), jnp.float32),
        scratch_types=[
            pltpu.VMEM((B_PER_W,), jnp.int32),
            pltpu.VMEM((HALF, 2 * D), jnp.float32),
            pltpu.SemaphoreType.DMA,
        ],
    )
    def gather_kernel(idx_hbm, wc_hbm, g_hbm, idx_v, rows_v, sem):
        wid = lax.axis_index("s") * NC + lax.axis_index("c")
        base = wid * B_PER_W
        pltpu.sync_copy(idx_hbm.at[pl.ds(base, B_PER_W)], idx_v)
        for h in range(2):
            copies = []
            for lo in range(0, HALF, CH):
                sz = min(CH, HALF - lo)
                copies.append(pltpu.async_copy(
                    wc_hbm.at[idx_v.at[pl.ds(h * HALF + lo, sz)]],
                    rows_v.at[pl.ds(lo, sz)],
                    sem,
                ))
            for cp in copies:
                cp.wait()
            pltpu.sync_copy(rows_v, g_hbm.at[pl.ds(base + h * HALF, HALF)])

    return gather_kernel(idx_flat, wc)


def _tc_combine(g3, uft):
    tt = 5

    def body(g_ref, u_ref, out_ref):
        for t in range(tt):
            g = g_ref[t]
            for m in range(M):
                out_ref[m, t] = lax.dot_general(
                    u_ref[m], g, (((1,), (1,)), ((), ())),
                    preferred_element_type=jnp.float32)

    return pl.pallas_call(
        body,
        grid=(T // tt,),
        in_specs=[
            pl.BlockSpec((tt, B, 2 * D), lambda i: (i, 0, 0)),
            pl.BlockSpec((M, D, 2 * D), lambda i: (0, 0, 0)),
        ],
        out_specs=pl.BlockSpec((M, tt, D, B), lambda i: (0, i, 0, 0)),
        out_shape=jax.ShapeDtypeStruct((M, T, D, B), jnp.float32),
    )(g3, uft)


def kernel(x, weight, lora_A, lora_B):
    idx_flat = jnp.swapaxes(x, 0, 1).reshape(N)
    wc = _tc_prep(jnp.transpose(weight), lora_A.reshape(M * R, V))
    g = _sc_gather(idx_flat, wc)

    p = SCALING * jnp.transpose(lora_B, (0, 2, 1))
    p_tiled = jnp.tile(p, (1, M, 1))
    sel = (jnp.arange(M * R)[None, :, None] // R
           == jnp.arange(M)[:, None, None])
    p_big = jnp.where(sel, p_tiled, 0.0)
    eye = jnp.broadcast_to(jnp.eye(D, dtype=jnp.float32), (M, D, D))
    u = jnp.concatenate([eye, p_big], axis=1)
    uft = jnp.transpose(u, (0, 2, 1))

    out_phys = _tc_combine(g.reshape(T, B, 2 * D), uft)
    return jnp.transpose(out_phys, (0, 3, 1, 2))


# speedup vs baseline: 3.9631x; 1.0466x over previous
import functools

import jax
import jax.numpy as jnp
from jax import lax
from jax.experimental import pallas as pl
from jax.experimental.pallas import tpu as pltpu
from jax.experimental.pallas import tpu_sc as plsc

M = 8
R = 8
V = 100000
D = 64
B = 1024
T = 50
LORA_ALPHA = 16.0
SCALING = LORA_ALPHA / R

N = B * T
NC, NS = 2, 16
NW = NC * NS
B_PER_W = N // NW
HALF = B_PER_W // 2
CH = 128


def _tc_prep(weight, lora_a_flat):
    vt = 8192
    grid = (pl.cdiv(V, vt),)

    def body(w_ref, a_ref, eye_ref, out_ref):
        stacked = jnp.concatenate([w_ref[...], a_ref[...]], axis=0)
        out_ref[...] = lax.dot_general(
            stacked, eye_ref[...], (((0,), (0,)), ((), ())),
            preferred_element_type=jnp.float32)

    return pl.pallas_call(
        body,
        grid=grid,
        in_specs=[
            pl.BlockSpec((D, vt), lambda i: (0, i)),
            pl.BlockSpec((M * R, vt), lambda i: (0, i)),
            pl.BlockSpec((2 * D, 2 * D), lambda i: (0, 0)),
        ],
        out_specs=pl.BlockSpec((vt, 2 * D), lambda i: (i, 0)),
        out_shape=jax.ShapeDtypeStruct((V, 2 * D), jnp.float32),
    )(weight, lora_a_flat, jnp.eye(2 * D, dtype=jnp.float32))


def _sc_gather(idx_flat, wc):
    mesh = plsc.VectorSubcoreMesh(core_axis_name="c", subcore_axis_name="s")

    @functools.partial(
        pl.kernel,
        mesh=mesh,
        compiler_params=pltpu.CompilerParams(use_tc_tiling_on_sc=True),
        out_type=jax.ShapeDtypeStruct((N, 2 * D), jnp.float32),
        scratch_types=[
            pltpu.VMEM((B_PER_W,), jnp.int32),
            pltpu.VMEM((HALF, 2 * D), jnp.float32),
            pltpu.SemaphoreType.DMA,
        ],
    )
    def gather_kernel(idx_hbm, wc_hbm, g_hbm, idx_v, rows_v, sem):
        wid = lax.axis_index("s") * NC + lax.axis_index("c")
        base = wid * B_PER_W
        pltpu.sync_copy(idx_hbm.at[pl.ds(base, B_PER_W)], idx_v)
        for h in range(2):
            copies = []
            for lo in range(0, HALF, CH):
                sz = min(CH, HALF - lo)
                copies.append(pltpu.async_copy(
                    wc_hbm.at[idx_v.at[pl.ds(h * HALF + lo, sz)]],
                    rows_v.at[pl.ds(lo, sz)],
                    sem,
                ))
            for cp in copies:
                cp.wait()
            pltpu.sync_copy(rows_v, g_hbm.at[pl.ds(base + h * HALF, HALF)])

    return gather_kernel(idx_flat, wc)


def _tc_combine(g3, uft):
    tt = 5

    def body(g_ref, u_ref, out_ref):
        for t in range(tt):
            g = g_ref[t]
            for m in range(M):
                out_ref[m, t] = lax.dot_general(
                    u_ref[m], g, (((1,), (1,)), ((), ())),
                    preferred_element_type=jnp.float32)

    return pl.pallas_call(
        body,
        grid=(T // tt,),
        in_specs=[
            pl.BlockSpec((tt, B, 2 * D), lambda i: (i, 0, 0)),
            pl.BlockSpec((M, D, 2 * D), lambda i: (0, 0, 0)),
        ],
        out_specs=pl.BlockSpec((M, tt, D, B), lambda i: (0, i, 0, 0)),
        out_shape=jax.ShapeDtypeStruct((M, T, D, B), jnp.float32),
    )(g3, uft)


def kernel(x, weight, lora_A, lora_B):
    idx_flat = jnp.swapaxes(x, 0, 1).reshape(N)
    wc = _tc_prep(jnp.transpose(weight), lora_A.reshape(M * R, V))
    g = _sc_gather(idx_flat, wc)

    p = SCALING * jnp.transpose(lora_B, (0, 2, 1))
    p_tiled = jnp.tile(p, (1, M, 1))
    sel = (jnp.arange(M * R)[None, :, None] // R
           == jnp.arange(M)[:, None, None])
    p_big = jnp.where(sel, p_tiled, 0.0)
    eye = jnp.broadcast_to(jnp.eye(D, dtype=jnp.float32), (M, D, D))
    u = jnp.concatenate([eye, p_big], axis=1)
    uft = jnp.transpose(u, (0, 2, 1))

    out_phys = _tc_combine(g.reshape(T, B, 2 * D), uft)
    return jnp.transpose(out_phys, (0, 3, 1, 2))
